# baseline (speedup 1.0000x reference)
import functools

import jax
import jax.numpy as jnp
from jax import lax
from jax.experimental import pallas as pl
from jax.experimental.pallas import tpu as pltpu
from jax.experimental.pallas import tpu_sc as plsc

VOCAB = 10
DIM = 10
TOK = 200
ROWS = 16384
N = TOK * ROWS
NW = 32
PW = N // NW
K = 6400
NCH = PW // K


def _sc_body(x_hbm, w1_hbm, w2_hbm, out_hbm, x_v, w1_v, w2_v, lut_v, out_v):
    wid = lax.axis_index("s") * 2 + lax.axis_index("c")
    base = wid * PW

    pltpu.sync_copy(w1_hbm, w1_v)
    pltpu.sync_copy(w2_hbm, w2_v)
    vi = lax.iota(jnp.int32, 16)
    msk = vi < VOCAB
    vc = jnp.where(msk, vi, 0)
    for d in range(DIM):
        addr = vc * DIM + d
        a = plsc.load_gather(w1_v, [addr], mask=msk)
        b = plsc.load_gather(w2_v, [addr], mask=msk)
        lut_v[pl.ds(d * 16, 16)] = a + b

    for it in range(NCH):
        off = base + it * K
        pltpu.sync_copy(x_hbm.at[pl.ds(off, K)], x_v)

        def body(j, _):
            xv = x_v[pl.ds(j * 16, 16)]
            for d in range(DIM):
                addr = xv + (16 * d)
                val = plsc.load_gather(lut_v, [addr])
                out_v[d, pl.ds(j * 16, 16)] = val
            return 0

        lax.fori_loop(0, K // 16, body, 0)
        for d in range(DIM):
            pltpu.sync_copy(out_v.at[d], out_hbm.at[d, pl.ds(off, K)])


@jax.jit
def kernel(x, W1, W2):
    xflat = x.T.reshape(N)
    mesh = plsc.VectorSubcoreMesh(core_axis_name="c", subcore_axis_name="s")
    outflat = pl.kernel(
        _sc_body,
        mesh=mesh,
        compiler_params=pltpu.CompilerParams(needs_layout_passes=False),
        out_type=jax.ShapeDtypeStruct((DIM, N), jnp.float32),
        scratch_types=[
            pltpu.VMEM((K,), jnp.int32),
            pltpu.VMEM((VOCAB * DIM,), jnp.float32),
            pltpu.VMEM((VOCAB * DIM,), jnp.float32),
            pltpu.VMEM((16 * DIM,), jnp.float32),
            pltpu.VMEM((DIM, K), jnp.float32),
        ],
    )(xflat, W1.reshape(VOCAB * DIM), W2.reshape(VOCAB * DIM))
    return outflat.reshape(DIM, TOK, ROWS).transpose(2, 1, 0)


# speedup vs baseline: 11.1517x; 11.1517x over previous
import jax
import jax.numpy as jnp
from jax.experimental import pallas as pl
from jax.experimental.pallas import tpu as pltpu

VOCAB = 10
DIM = 10
TOK = 200
ROWS = 16384
BT = 8
BC = 16384


CH = 512


def _body(x_ref, w1_ref, w2_ref, out_ref):
    ws = [[w1_ref[v, d] + w2_ref[v, d] for d in range(DIM)]
          for v in range(VOCAB)]
    for c in range(BC // CH):
        sl = slice(c * CH, (c + 1) * CH)
        xc = x_ref[:, sl]
        b0 = (xc & 1) != 0
        b1 = (xc & 2) != 0
        b2 = (xc & 4) != 0
        b3 = xc >= 8
        for d in range(DIM):
            s01 = jnp.where(b0, ws[1][d], ws[0][d])
            s23 = jnp.where(b0, ws[3][d], ws[2][d])
            s45 = jnp.where(b0, ws[5][d], ws[4][d])
            s67 = jnp.where(b0, ws[7][d], ws[6][d])
            s89 = jnp.where(b0, ws[9][d], ws[8][d])
            t03 = jnp.where(b1, s23, s01)
            t47 = jnp.where(b1, s67, s45)
            u07 = jnp.where(b2, t47, t03)
            out_ref[d, :, sl] = jnp.where(b3, s89, u07)


@jax.jit
def kernel(x, W1, W2):
    xt = x.T
    outt = pl.pallas_call(
        _body,
        grid=(TOK // BT, ROWS // BC),
        in_specs=[
            pl.BlockSpec((BT, BC), lambda i, j: (i, j)),
            pl.BlockSpec(memory_space=pltpu.SMEM),
            pl.BlockSpec(memory_space=pltpu.SMEM),
        ],
        out_specs=pl.BlockSpec((DIM, BT, BC), lambda i, j: (0, i, j)),
        out_shape=jax.ShapeDtypeStruct((DIM, TOK, ROWS), jnp.float32),
    )(xt, W1, W2)
    return outt.transpose(2, 1, 0)
